# Initial kernel scaffold; baseline (speedup 1.0000x reference)
import jax
import jax.numpy as jnp
from jax.experimental import pallas as pl


def kernel(x, edge_index, params):
    raise NotImplementedError("write your pallas kernel here")


# speedup vs baseline: 11.9031x; 11.9031x over previous
import functools

import jax
import jax.numpy as jnp
from jax import lax
from jax.experimental import pallas as pl
from jax.experimental.pallas import tpu as pltpu
from jax.experimental.pallas import tpu_sc as plsc

N = 10000
E = 160000
IN_DIM = 256
HID = 512
HEADS = 8
HD = 64

R = 256
NP = 10240
GRID = NP // R
DUMMY = N

NSC = 2
NTILE = 16
CH = 128
EP = 163840
E_PER_SC = EP // NSC
E_PER_TILE = E_PER_SC // NTILE
NCH = E_PER_TILE // CH
ROWS_PER_TILE = NP // NTILE
ZROWS = 40

f32 = jnp.float32


def _k0_body(x_ref, w_ref, b_ref, o_ref):
    o_ref[...] = (
        jnp.dot(x_ref[...], w_ref[...], preferred_element_type=f32) + b_ref[...]
    )


def _k0(xp, in_w, in_b):
    return pl.pallas_call(
        _k0_body,
        grid=(GRID,),
        in_specs=[
            pl.BlockSpec((R, IN_DIM), lambda i: (i, 0)),
            pl.BlockSpec((IN_DIM, HID), lambda i: (0, 0)),
            pl.BlockSpec((1, HID), lambda i: (0, 0)),
        ],
        out_specs=pl.BlockSpec((R, HID), lambda i: (i, 0)),
        out_shape=jax.ShapeDtypeStruct((NP, HID), f32),
    )(xp, in_w, in_b)


def _k1_body(x_ref, w_ref, am_ref, h0_ref, h1_ref, h2_ref, h3_ref, al_ref,
             smax_ref):
    h = jnp.dot(x_ref[...], w_ref[...], preferred_element_type=f32)
    h0_ref[...] = h[:, 0:128]
    h1_ref[...] = h[:, 128:256]
    h2_ref[...] = h[:, 256:384]
    h3_ref[...] = h[:, 384:512]
    al = jnp.dot(h, am_ref[...], preferred_element_type=f32)
    al_ref[...] = al
    bm = jnp.max(al, axis=0, keepdims=True)
    i = pl.program_id(0)

    @pl.when(i == 0)
    def _():
        smax_ref[...] = bm

    @pl.when(i > 0)
    def _():
        smax_ref[...] = jnp.maximum(smax_ref[...], bm)


def _k1(x_in, gat_w, am):
    return pl.pallas_call(
        _k1_body,
        grid=(GRID,),
        in_specs=[
            pl.BlockSpec((R, HID), lambda i: (i, 0)),
            pl.BlockSpec((HID, HID), lambda i: (0, 0)),
            pl.BlockSpec((HID, 16), lambda i: (0, 0)),
        ],
        out_specs=[
            pl.BlockSpec((R, 128), lambda i: (i, 0)),
            pl.BlockSpec((R, 128), lambda i: (i, 0)),
            pl.BlockSpec((R, 128), lambda i: (i, 0)),
            pl.BlockSpec((R, 128), lambda i: (i, 0)),
            pl.BlockSpec((R, 16), lambda i: (i, 0)),
            pl.BlockSpec((1, 16), lambda i: (0, 0)),
        ],
        out_shape=[
            jax.ShapeDtypeStruct((NP, 128), f32),
            jax.ShapeDtypeStruct((NP, 128), f32),
            jax.ShapeDtypeStruct((NP, 128), f32),
            jax.ShapeDtypeStruct((NP, 128), f32),
            jax.ShapeDtypeStruct((NP, 16), f32),
            jax.ShapeDtypeStruct((1, 16), f32),
        ],
    )(x_in, gat_w, am)


def _lrelu(t):
    return jnp.maximum(t, 0.2 * t)


def _k2_body(with_mem, al_ref, smax_ref, x_ref, m_ref, uw_ref, ub1_ref,
             nw_ref, nb_ref, asrc_ref, adst_ref, s_ref, exs_ref, preu_ref,
             pren_ref):
    al = al_ref[...]
    a_s = al[:, :8]
    a_d = al[:, 8:]
    sm = smax_ref[...][:, :8]
    s = _lrelu(sm + a_d)
    exs_ref[...] = jnp.exp(_lrelu(a_s + a_d) - s)
    asrc_ref[...] = jnp.concatenate([a_s, a_s], axis=1)
    adst_ref[...] = jnp.concatenate([a_d, a_d], axis=1)
    s_ref[...] = jnp.concatenate([s, s], axis=1)
    if with_mem:
        xm = jnp.concatenate([x_ref[...], m_ref[...]], axis=1)
    else:
        xm = x_ref[...]
    preu_ref[...] = (
        jnp.dot(xm, uw_ref[...], preferred_element_type=f32) + ub1_ref[...]
    )
    pren_ref[...] = (
        jnp.dot(xm, nw_ref[...], preferred_element_type=f32) + nb_ref[...]
    )


def _k2(al, smax, x_in, mem, u_wxm, u_b1, n_wxm, n_b, with_mem):
    kdim = 2 * HID if with_mem else HID
    return pl.pallas_call(
        functools.partial(_k2_body, with_mem),
        grid=(GRID,),
        in_specs=[
            pl.BlockSpec((R, 16), lambda i: (i, 0)),
            pl.BlockSpec((1, 16), lambda i: (0, 0)),
            pl.BlockSpec((R, HID), lambda i: (i, 0)),
            pl.BlockSpec((R, HID), lambda i: (i, 0)),
            pl.BlockSpec((kdim, 2 * HID), lambda i: (0, 0)),
            pl.BlockSpec((1, 2 * HID), lambda i: (0, 0)),
            pl.BlockSpec((kdim, HID), lambda i: (0, 0)),
            pl.BlockSpec((1, HID), lambda i: (0, 0)),
        ],
        out_specs=[
            pl.BlockSpec((R, 16), lambda i: (i, 0)),
            pl.BlockSpec((R, 16), lambda i: (i, 0)),
            pl.BlockSpec((R, 16), lambda i: (i, 0)),
            pl.BlockSpec((R, 8), lambda i: (i, 0)),
            pl.BlockSpec((R, 2 * HID), lambda i: (i, 0)),
            pl.BlockSpec((R, HID), lambda i: (i, 0)),
        ],
        out_shape=[
            jax.ShapeDtypeStruct((NP, 16), f32),
            jax.ShapeDtypeStruct((NP, 16), f32),
            jax.ShapeDtypeStruct((NP, 16), f32),
            jax.ShapeDtypeStruct((NP, 8), f32),
            jax.ShapeDtypeStruct((NP, 2 * HID), f32),
            jax.ShapeDtypeStruct((NP, HID), f32),
        ],
    )(al, smax, x_in, mem, u_wxm, u_b1, n_wxm, n_b)


def _k2b_body(al_ref, smax_ref, asrc_ref, adst_ref, s_ref, exs_ref):
    al = al_ref[...]
    a_s = al[:, :8]
    a_d = al[:, 8:]
    sm = smax_ref[...][:, :8]
    s = _lrelu(sm + a_d)
    exs_ref[...] = jnp.exp(_lrelu(a_s + a_d) - s)
    asrc_ref[...] = jnp.concatenate([a_s, a_s], axis=1)
    adst_ref[...] = jnp.concatenate([a_d, a_d], axis=1)
    s_ref[...] = jnp.concatenate([s, s], axis=1)


def _k2b(al, smax):
    return pl.pallas_call(
        _k2b_body,
        grid=(GRID,),
        in_specs=[
            pl.BlockSpec((R, 16), lambda i: (i, 0)),
            pl.BlockSpec((1, 16), lambda i: (0, 0)),
        ],
        out_specs=[
            pl.BlockSpec((R, 16), lambda i: (i, 0)),
            pl.BlockSpec((R, 16), lambda i: (i, 0)),
            pl.BlockSpec((R, 16), lambda i: (i, 0)),
            pl.BlockSpec((R, 8), lambda i: (i, 0)),
        ],
        out_shape=[
            jax.ShapeDtypeStruct((NP, 16), f32),
            jax.ShapeDtypeStruct((NP, 16), f32),
            jax.ShapeDtypeStruct((NP, 16), f32),
            jax.ShapeDtypeStruct((NP, 8), f32),
        ],
    )(al, smax)


def _gat_out(acc_ref, den_ref, exs_ref, h0_ref, h1_ref, h2_ref, h3_ref,
             x_ref, gatb_ref, lng_ref, lnb_ref):
    acc = acc_ref[...]
    a = acc[0] + acc[1]
    agg = jnp.concatenate([a[0], a[1], a[2], a[3]], axis=1)
    den = den_ref[...]
    d8 = den[0][:, :8] + den[1][:, :8]
    exs = exs_ref[...]
    h = jnp.concatenate(
        [h0_ref[...], h1_ref[...], h2_ref[...], h3_ref[...]], axis=1)
    dt = d8 + exs + 1e-16
    num = agg.reshape(R, HEADS, HD) + exs[:, :, None] * h.reshape(R, HEADS, HD)
    out = (num / dt[:, :, None]).reshape(R, HID)
    g0 = out + gatb_ref[...] + x_ref[...]
    mu = jnp.mean(g0, axis=1, keepdims=True)
    var = jnp.mean((g0 - mu) ** 2, axis=1, keepdims=True)
    return (g0 - mu) * lax.rsqrt(var + 1e-5) * lng_ref[...] + lnb_ref[...]


def _k3_body(acc_ref, den_ref, exs_ref, h0_ref, h1_ref, h2_ref, h3_ref,
             x_ref, m_ref, preu_ref, pren_ref, gatb_ref, lng_ref, lnb_ref,
             uwg_ref, uw2_ref, ub2_ref, nwg_ref, g_ref, mem_ref):
    g = _gat_out(acc_ref, den_ref, exs_ref, h0_ref, h1_ref, h2_ref, h3_ref,
                 x_ref, gatb_ref, lng_ref, lnb_ref)
    g_ref[...] = g
    t1 = jnp.maximum(
        preu_ref[...] + jnp.dot(g, uwg_ref[...], preferred_element_type=f32),
        0.0)
    u = jax.nn.sigmoid(
        jnp.dot(t1, uw2_ref[...], preferred_element_type=f32) + ub2_ref[...])
    n = jnp.tanh(
        pren_ref[...] + jnp.dot(g, nwg_ref[...], preferred_element_type=f32))
    mem_ref[...] = u * m_ref[...] + (1.0 - u) * n


def _k3(acc, den, exs, h2p, x_in, mem, preu, pren, gatb, lng, lnb, u_wg,
        u_w2, u_b2, n_wg):
    return pl.pallas_call(
        _k3_body,
        grid=(GRID,),
        in_specs=[
            pl.BlockSpec((2, 4, R, 128), lambda i: (0, 0, i, 0)),
            pl.BlockSpec((2, R, 16), lambda i: (0, i, 0)),
            pl.BlockSpec((R, 8), lambda i: (i, 0)),
            pl.BlockSpec((R, 128), lambda i: (i, 0)),
            pl.BlockSpec((R, 128), lambda i: (i, 0)),
            pl.BlockSpec((R, 128), lambda i: (i, 0)),
            pl.BlockSpec((R, 128), lambda i: (i, 0)),
            pl.BlockSpec((R, HID), lambda i: (i, 0)),
            pl.BlockSpec((R, HID), lambda i: (i, 0)),
            pl.BlockSpec((R, 2 * HID), lambda i: (i, 0)),
            pl.BlockSpec((R, HID), lambda i: (i, 0)),
            pl.BlockSpec((1, HID), lambda i: (0, 0)),
            pl.BlockSpec((1, HID), lambda i: (0, 0)),
            pl.BlockSpec((1, HID), lambda i: (0, 0)),
            pl.BlockSpec((HID, 2 * HID), lambda i: (0, 0)),
            pl.BlockSpec((2 * HID, HID), lambda i: (0, 0)),
            pl.BlockSpec((1, HID), lambda i: (0, 0)),
            pl.BlockSpec((HID, HID), lambda i: (0, 0)),
        ],
        out_specs=[
            pl.BlockSpec((R, HID), lambda i: (i, 0)),
            pl.BlockSpec((R, HID), lambda i: (i, 0)),
        ],
        out_shape=[
            jax.ShapeDtypeStruct((NP, HID), f32),
            jax.ShapeDtypeStruct((NP, HID), f32),
        ],
    )(acc, den, exs, h2p[0], h2p[1], h2p[2], h2p[3], x_in, mem, preu, pren,
      gatb, lng, lnb, u_wg, u_w2, u_b2, n_wg)


def _k3b_body(acc_ref, den_ref, exs_ref, h0_ref, h1_ref, h2_ref, h3_ref,
              x_ref, gatb_ref, lng_ref, lnb_ref, g_ref):
    g_ref[...] = _gat_out(acc_ref, den_ref, exs_ref, h0_ref, h1_ref, h2_ref,
                          h3_ref, x_ref, gatb_ref, lng_ref, lnb_ref)


def _k3b(acc, den, exs, h2p, x_in, gatb, lng, lnb):
    return pl.pallas_call(
        _k3b_body,
        grid=(GRID,),
        in_specs=[
            pl.BlockSpec((2, 4, R, 128), lambda i: (0, 0, i, 0)),
            pl.BlockSpec((2, R, 16), lambda i: (0, i, 0)),
            pl.BlockSpec((R, 8), lambda i: (i, 0)),
            pl.BlockSpec((R, 128), lambda i: (i, 0)),
            pl.BlockSpec((R, 128), lambda i: (i, 0)),
            pl.BlockSpec((R, 128), lambda i: (i, 0)),
            pl.BlockSpec((R, 128), lambda i: (i, 0)),
            pl.BlockSpec((R, HID), lambda i: (i, 0)),
            pl.BlockSpec((1, HID), lambda i: (0, 0)),
            pl.BlockSpec((1, HID), lambda i: (0, 0)),
            pl.BlockSpec((1, HID), lambda i: (0, 0)),
        ],
        out_specs=pl.BlockSpec((R, HID), lambda i: (i, 0)),
        out_shape=jax.ShapeDtypeStruct((NP, HID), f32),
    )(acc, den, exs, h2p[0], h2p[1], h2p[2], h2p[3], x_in, gatb, lng, lnb)


def _k5_body(h_ref, gpw_ref, gpb_ref, cw1_ref, cb1_ref, cw2_ref, cb2_ref,
             out_ref, ssum, smaxp):
    i = pl.program_id(0)
    blk = h_ref[...]
    rows = i * R + lax.broadcasted_iota(jnp.int32, (R, 1), 0)
    valid = rows < N
    bsum = jnp.sum(jnp.where(valid, blk, 0.0), axis=0, keepdims=True)
    bmax = jnp.max(jnp.where(valid, blk, -3.0e38), axis=0, keepdims=True)

    @pl.when(i == 0)
    def _():
        ssum[...] = bsum
        smaxp[...] = bmax

    @pl.when(i > 0)
    def _():
        ssum[...] = ssum[...] + bsum
        smaxp[...] = jnp.maximum(smaxp[...], bmax)

    @pl.when(i == GRID - 1)
    def _():
        total = ssum[...]
        pooled = jnp.concatenate([total / N, smaxp[...], total], axis=1)
        t = jnp.maximum(
            jnp.dot(pooled, gpw_ref[...], preferred_element_type=f32)
            + gpb_ref[...], 0.0)
        t2 = jnp.maximum(
            jnp.dot(t, cw1_ref[...], preferred_element_type=f32)
            + cb1_ref[...], 0.0)
        out_ref[...] = (
            jnp.dot(t2, cw2_ref[...], preferred_element_type=f32)
            + cb2_ref[...])


def _k5(h, gp_w, gp_b, c_w1, c_b1, c_w2, c_b2):
    return pl.pallas_call(
        _k5_body,
        grid=(GRID,),
        in_specs=[
            pl.BlockSpec((R, HID), lambda i: (i, 0)),
            pl.BlockSpec((3 * HID, HID), lambda i: (0, 0)),
            pl.BlockSpec((1, HID), lambda i: (0, 0)),
            pl.BlockSpec((HID, HID // 2), lambda i: (0, 0)),
            pl.BlockSpec((1, HID // 2), lambda i: (0, 0)),
            pl.BlockSpec((HID // 2, 1), lambda i: (0, 0)),
            pl.BlockSpec((1, 1), lambda i: (0, 0)),
        ],
        out_specs=pl.BlockSpec((1, 1), lambda i: (0, 0)),
        out_shape=jax.ShapeDtypeStruct((1, 1), f32),
        scratch_shapes=[
            pltpu.VMEM((1, HID), f32),
            pltpu.VMEM((1, HID), f32),
        ],
    )(h, gp_w, gp_b, c_w1, c_b1, c_w2, c_b2)


def _sc1_body(src_hbm, dst_hbm, asrc_hbm, adst_hbm, s_hbm, ex_hbm, den_hbm,
              srcv, dstv, ga, gb, gs, exv, zbuf, densh, sem1, sem2, sem3):
    c = lax.axis_index("c")
    sid = lax.axis_index("s")

    def zrow(k, carry):
        zbuf[k, :] = jnp.zeros((16,), f32)
        return carry

    lax.fori_loop(0, ROWS_PER_TILE, zrow, 0)
    pltpu.sync_copy(zbuf, densh.at[pl.ds(sid * ROWS_PER_TILE, ROWS_PER_TILE)])
    plsc.subcore_barrier()

    base = c * E_PER_SC + sid * E_PER_TILE

    def chunk(ci, carry):
        e0 = base + ci * CH
        pltpu.sync_copy(src_hbm.at[pl.ds(e0, CH)], srcv)
        pltpu.sync_copy(dst_hbm.at[pl.ds(e0, CH)], dstv)
        cp1 = pltpu.async_copy(asrc_hbm.at[srcv], ga, sem1)
        cp2 = pltpu.async_copy(adst_hbm.at[dstv], gb, sem2)
        cp3 = pltpu.async_copy(s_hbm.at[dstv], gs, sem3)
        cp1.wait()
        cp2.wait()
        cp3.wait()

        def per_edge(k, cc):
            t = ga[k, :] + gb[k, :]
            exv[k, :] = jnp.exp(jnp.maximum(t, 0.2 * t) - gs[k, :])
            return cc

        lax.fori_loop(0, CH, per_edge, 0)
        pltpu.sync_copy(exv, ex_hbm.at[pl.ds(e0, CH)])
        pltpu.sync_copy(exv, densh.at[dstv], add=True)
        return carry

    lax.fori_loop(0, NCH, chunk, 0)
    plsc.subcore_barrier()
    pltpu.sync_copy(densh.at[pl.ds(sid * ROWS_PER_TILE, ROWS_PER_TILE)],
                    den_hbm.at[c, pl.ds(sid * ROWS_PER_TILE, ROWS_PER_TILE)])


def _build_sc1(interpret=False):
    return pl.kernel(
        _sc1_body,
        out_type=[
            jax.ShapeDtypeStruct((EP, 16), f32),
            jax.ShapeDtypeStruct((NSC, NP, 16), f32),
        ],
        mesh=plsc.VectorSubcoreMesh(core_axis_name="c", subcore_axis_name="s",
                                    num_cores=NSC, num_subcores=NTILE),
        compiler_params=pltpu.CompilerParams(use_tc_tiling_on_sc=False),
        scratch_types=[
            pltpu.VMEM((CH,), jnp.int32),
            pltpu.VMEM((CH,), jnp.int32),
            pltpu.VMEM((CH, 16), f32),
            pltpu.VMEM((CH, 16), f32),
            pltpu.VMEM((CH, 16), f32),
            pltpu.VMEM((CH, 16), f32),
            pltpu.VMEM((ROWS_PER_TILE, 16), f32),
            pltpu.VMEM_SHARED((NP, 16), f32),
            pltpu.SemaphoreType.DMA,
            pltpu.SemaphoreType.DMA,
            pltpu.SemaphoreType.DMA,
        ],
        interpret=interpret,
    )


def _sc2_body(src_hbm, dst_hbm, ex_hbm, h0_hbm, h1_hbm, h2_hbm, h3_hbm,
              acc_hbm, srcv, dstv, exv, rows, zbuf, accsh, semg):
    c = lax.axis_index("c")
    sid = lax.axis_index("s")

    def zrow(k, carry):
        for j in range(8):
            zbuf[k, pl.ds(16 * j, 16)] = jnp.zeros((16,), f32)
        return carry

    lax.fori_loop(0, ZROWS, zrow, 0)

    base = c * E_PER_SC + sid * E_PER_TILE
    tables = [h0_hbm, h1_hbm, h2_hbm, h3_hbm]
    for p in range(4):
        def zcp(j, carry):
            pltpu.sync_copy(
                zbuf, accsh.at[pl.ds(sid * ROWS_PER_TILE + j * ZROWS, ZROWS)])
            return carry

        lax.fori_loop(0, ROWS_PER_TILE // ZROWS, zcp, 0)
        plsc.subcore_barrier()

        def chunk(ci, carry, p=p):
            e0 = base + ci * CH
            pltpu.sync_copy(src_hbm.at[pl.ds(e0, CH)], srcv)
            pltpu.sync_copy(dst_hbm.at[pl.ds(e0, CH)], dstv)
            pltpu.sync_copy(ex_hbm.at[pl.ds(e0, CH)], exv)
            pltpu.async_copy(tables[p].at[srcv], rows, semg).wait()

            def per_edge(k, cc, p=p):
                row = exv[k, :]
                va = jnp.full((16,), row[2 * p], f32)
                vb = jnp.full((16,), row[2 * p + 1], f32)
                for j in range(4):
                    rows[k, pl.ds(16 * j, 16)] = rows[k, pl.ds(16 * j, 16)] * va
                for j in range(4, 8):
                    rows[k, pl.ds(16 * j, 16)] = rows[k, pl.ds(16 * j, 16)] * vb
                return cc

            lax.fori_loop(0, CH, per_edge, 0)
            pltpu.sync_copy(rows, accsh.at[dstv], add=True)
            return carry

        lax.fori_loop(0, NCH, chunk, 0)
        plsc.subcore_barrier()
        pltpu.sync_copy(accsh.at[pl.ds(sid * ROWS_PER_TILE, ROWS_PER_TILE)],
                        acc_hbm.at[c, p,
                                   pl.ds(sid * ROWS_PER_TILE, ROWS_PER_TILE)])
        plsc.subcore_barrier()


def _build_sc2(interpret=False):
    return pl.kernel(
        _sc2_body,
        out_type=jax.ShapeDtypeStruct((NSC, 4, NP, 128), f32),
        mesh=plsc.VectorSubcoreMesh(core_axis_name="c", subcore_axis_name="s",
                                    num_cores=NSC, num_subcores=NTILE),
        compiler_params=pltpu.CompilerParams(use_tc_tiling_on_sc=False),
        scratch_types=[
            pltpu.VMEM((CH,), jnp.int32),
            pltpu.VMEM((CH,), jnp.int32),
            pltpu.VMEM((CH, 16), f32),
            pltpu.VMEM((CH, 128), f32),
            pltpu.VMEM((ZROWS, 128), f32),
            pltpu.VMEM_SHARED((NP, 128), f32),
            pltpu.SemaphoreType.DMA,
        ],
        interpret=interpret,
    )


_sc1 = None
_sc2 = None


def _get_sc_kernels():
    global _sc1, _sc2
    if _sc1 is None:
        _sc1 = _build_sc1()
    if _sc2 is None:
        _sc2 = _build_sc2()
    return _sc1, _sc2


def _head_mat(att):
    eye = jnp.eye(HEADS, dtype=f32)
    return (eye[:, None, :] * att[:, :, None]).reshape(HID, HEADS)


def kernel(x, edge_index, params):
    xp = jnp.pad(x, ((0, NP - N), (0, 0)))
    pad = jnp.full((EP - E,), DUMMY, jnp.int32)
    srcp = jnp.concatenate([edge_index[0].astype(jnp.int32), pad])
    dstp = jnp.concatenate([edge_index[1].astype(jnp.int32), pad])

    h = _k0(xp, params['in_W'], params['in_b'].reshape(1, HID))
    mem = jnp.zeros((NP, HID), f32)

    for i in range(3):
        p = params['layer%d' % i]
        am = jnp.concatenate(
            [_head_mat(p['att_src']), _head_mat(p['att_dst'])], axis=1)
        h2p0, h2p1, h2p2, h2p3, al, smax = _k1(h, p['gat_W'], am)
        h2p = (h2p0, h2p1, h2p2, h2p3)
        last = (i == 2)
        if not last:
            u_wxm = p['u_W1'][:2 * HID]
            u_wg = p['u_W1'][2 * HID:]
            n_wxm = p['n_W'][:2 * HID]
            n_wg = p['n_W'][2 * HID:]
            if i == 0:
                u_wxm = p['u_W1'][:HID]
                n_wxm = p['n_W'][:HID]
            asrc16, adst16, s16, exs, preu, pren = _k2(
                al, smax, h, mem, u_wxm, p['u_b1'].reshape(1, 2 * HID),
                n_wxm, p['n_b'].reshape(1, HID), with_mem=(i != 0))
        else:
            asrc16, adst16, s16, exs = _k2b(al, smax)

        sc1, sc2 = _get_sc_kernels()
        ex_e, den = sc1(srcp, dstp, asrc16, adst16, s16)
        acc = sc2(srcp, dstp, ex_e, *h2p)

        if not last:
            h, mem = _k3(acc, den, exs, h2p, h, mem, preu, pren,
                         p['gat_b'].reshape(1, HID),
                         p['ln_g'].reshape(1, HID),
                         p['ln_b'].reshape(1, HID),
                         u_wg, p['u_W2'], p['u_b2'].reshape(1, HID), n_wg)
        else:
            h = _k3b(acc, den, exs, h2p, h,
                     p['gat_b'].reshape(1, HID),
                     p['ln_g'].reshape(1, HID),
                     p['ln_b'].reshape(1, HID))

    return _k5(h, params['gp_W'], params['gp_b'].reshape(1, HID),
               params['c_W1'], params['c_b1'].reshape(1, HID // 2),
               params['c_W2'], params['c_b2'].reshape(1, 1))
